# baseline (speedup 1.0000x reference)
import functools

import jax
import jax.numpy as jnp
from jax import lax
from jax.experimental import pallas as pl
from jax.experimental.pallas import tpu as pltpu
from jax.experimental.pallas import tpu_sc as plsc

_N = 50000
_K = 32
_BLK = 384
_NFULL = _N // _BLK
_TAIL = _N - _NFULL * _BLK
_NW = 32
_TMAX = (_NFULL + _NW - 1) // _NW
_UMAX = (_TMAX + 1) // 2


@functools.lru_cache(maxsize=1)
def _sc_rowsum_call():
    mesh = plsc.VectorSubcoreMesh(core_axis_name="c", subcore_axis_name="s")

    @functools.partial(
        pl.kernel,
        mesh=mesh,
        out_type=jax.ShapeDtypeStruct((_N,), jnp.float32),
        scratch_types=[
            pltpu.VMEM((_K, _BLK), jnp.float32),
            pltpu.VMEM((_K, _BLK), jnp.float32),
            pltpu.VMEM((_K, _TAIL), jnp.float32),
            pltpu.VMEM((_BLK,), jnp.float32),
            pltpu.VMEM((16,), jnp.float32),
            pltpu.SemaphoreType.DMA,
            pltpu.SemaphoreType.DMA,
        ],
        compiler_params=pltpu.CompilerParams(skip_device_barrier=True),
    )
    def sc_rowsum(nst_hbm, tail_hbm, bias_hbm, out_hbm,
                  buf0, buf1, tbuf, obuf, bvec, sem0, sem1):
        wid = lax.axis_index("s") * 2 + lax.axis_index("c")
        pltpu.sync_copy(bias_hbm, bvec)

        def start(r, buf, sem):
            for c in range(_K // 8):
                pltpu.async_copy(
                    nst_hbm.at[pl.ds(8 * c, 8), pl.ds(r * _BLK, _BLK)],
                    buf.at[pl.ds(8 * c, 8)], sem)

        def wait(r, buf, sem):
            for c in range(_K // 8):
                pltpu.make_async_copy(
                    nst_hbm.at[pl.ds(8 * c, 8), pl.ds(r * _BLK, _BLK)],
                    buf.at[pl.ds(8 * c, 8)], sem
                ).wait()

        def reduce_to(src, g, dst_off):
            a0 = bvec[...]
            a1 = jnp.zeros((16,), jnp.float32)
            a2 = jnp.zeros((16,), jnp.float32)
            a3 = jnp.zeros((16,), jnp.float32)
            for js in range(0, _K, 4):
                a0 = a0 + src[js, pl.ds(g * 16, 16)]
                a1 = a1 + src[js + 1, pl.ds(g * 16, 16)]
                a2 = a2 + src[js + 2, pl.ds(g * 16, 16)]
                a3 = a3 + src[js + 3, pl.ds(g * 16, 16)]
            obuf[pl.ds(dst_off, 16)] = (a0 + a1) + (a2 + a3)

        def compute(buf, r):
            def grp(g, c2):
                reduce_to(buf, g, g * 16)
                return c2

            lax.fori_loop(0, _BLK // 16, grp, 0)
            pltpu.sync_copy(obuf, out_hbm.at[pl.ds(r * _BLK, _BLK)])

        start(wid, buf0, sem0)

        def dbl(u, carry):
            r0 = wid + _NW * 2 * u
            r1 = r0 + _NW
            r2 = r1 + _NW

            @pl.when(r1 < _NFULL)
            def _():
                start(r1, buf1, sem1)

            @pl.when(r0 < _NFULL)
            def _():
                wait(r0, buf0, sem0)
                compute(buf0, r0)

            @pl.when(r2 < _NFULL)
            def _():
                start(r2, buf0, sem0)

            @pl.when(r1 < _NFULL)
            def _():
                wait(r1, buf1, sem1)
                compute(buf1, r1)

            return carry

        lax.fori_loop(0, _UMAX, dbl, 0)

        @pl.when(wid == _NW - 1)
        def _():
            pltpu.sync_copy(tail_hbm, tbuf)

            def tgrp(g, c2):
                reduce_to(tbuf, g, g * 16)
                return c2

            lax.fori_loop(0, _TAIL // 16, tgrp, 0)
            pltpu.sync_copy(
                obuf.at[pl.ds(0, _TAIL)],
                out_hbm.at[pl.ds(_NFULL * _BLK, _TAIL)])

    return sc_rowsum


def kernel(query_emb, entity_emb, neighbor_scores, bias):
    del query_emb, entity_emb
    ns_t = neighbor_scores.T
    tail_t = jax.lax.slice(ns_t, (0, _NFULL * _BLK), (_K, _N))
    bias16 = jnp.broadcast_to(bias.astype(jnp.float32), (16,))
    return _sc_rowsum_call()(ns_t, tail_t, bias16)


# speedup vs baseline: 1.7657x; 1.7657x over previous
import jax
import jax.numpy as jnp
from jax.experimental import pallas as pl
from jax.experimental.pallas import tpu as pltpu

_N = 50000
_K = 32
_L = 2048


def _body(bias_ref, x_ref, o_ref):
    x = x_ref[...]
    o_ref[...] = jnp.sum(x[:_K, :], axis=0) + bias_ref[0]


def kernel(query_emb, entity_emb, neighbor_scores, bias):
    del query_emb, entity_emb
    ns_t = neighbor_scores.T
    return pl.pallas_call(
        _body,
        grid=((_N + _L - 1) // _L,),
        in_specs=[
            pl.BlockSpec(memory_space=pltpu.SMEM),
            pl.BlockSpec((ns_t.shape[0], _L), lambda i: (0, i)),
        ],
        out_specs=pl.BlockSpec((_L,), lambda i: (i,)),
        out_shape=jax.ShapeDtypeStruct((_N,), jnp.float32),
    )(bias, ns_t)
